# baseline (speedup 1.0000x reference)
import functools

import jax
import jax.numpy as jnp
from jax import lax
from jax.experimental import pallas as pl
from jax.experimental.pallas import tpu as pltpu
from jax.experimental.pallas import tpu_sc as plsc

N = 10000
C_IN = 128
C_OUT = 128
KV = 27
E = 324000
EPK = E // KV

CHUNK = 128
CPK = 94
EPK_PAD = CPK * CHUNK
NCH = KV * CPK

NC = 2
NS = 16
NW = NC * NS
NJ = 80
HJ = NJ // 2
NCHP = NW * NJ

ACC_ROWS = 10240
RPT = ACC_ROWS // NS


def _matmul_body(x_ref, w_ref, y_ref):
    y_ref[0] = jnp.dot(x_ref[...], w_ref[0],
                       preferred_element_type=jnp.float32)


def _all_offset_matmul(x, W):
    rb = 2000
    return pl.pallas_call(
        _matmul_body,
        grid=(N // rb, KV),
        in_specs=[
            pl.BlockSpec((rb, C_IN), lambda r, k: (r, 0)),
            pl.BlockSpec((1, C_IN, C_OUT), lambda r, k: (k, 0, 0)),
        ],
        out_specs=pl.BlockSpec((1, rb, C_OUT), lambda r, k: (k, r, 0)),
        out_shape=jax.ShapeDtypeStruct((KV, N, C_OUT), jnp.float32),
    )(x, W)


def _combine_body(a_ref, b_ref, o_ref):
    o_ref[...] = a_ref[...] + b_ref[...]


def _combine(p0, p1):
    rb = 2000
    return pl.pallas_call(
        _combine_body,
        grid=(N // rb,),
        in_specs=[
            pl.BlockSpec((rb, C_OUT), lambda r: (r, 0)),
            pl.BlockSpec((rb, C_OUT), lambda r: (r, 0)),
        ],
        out_specs=pl.BlockSpec((rb, C_OUT), lambda r: (r, 0)),
        out_shape=jax.ShapeDtypeStruct((N, C_OUT), jnp.float32),
    )(p0, p1)


def _sc_gather_scatter(y_flat, src2d, dst2d):
    mesh = plsc.VectorSubcoreMesh(core_axis_name="c", subcore_axis_name="s")

    @functools.partial(
        pl.kernel,
        mesh=mesh,
        out_type=jax.ShapeDtypeStruct((NC * ACC_ROWS, C_OUT), jnp.float32),
        scratch_types=[
            pltpu.VMEM((HJ, CHUNK), jnp.int32),
            pltpu.VMEM((HJ, CHUNK), jnp.int32),
            pltpu.VMEM((CHUNK, C_OUT), jnp.float32),
            pltpu.VMEM((CHUNK, C_OUT), jnp.float32),
            pltpu.VMEM_SHARED((ACC_ROWS, C_OUT), jnp.float32),
            pltpu.SemaphoreType.DMA,
            pltpu.SemaphoreType.DMA,
        ],
    )
    def sc_kernel(y_hbm, src_hbm, dst_hbm, out_hbm,
                  idx_s, idx_d, rows0, rows1, acc, ss0, ss1):
        cid = lax.axis_index("c")
        sid = lax.axis_index("s")
        tid = cid * NS + sid

        z16 = jnp.zeros((16,), jnp.float32)

        @pl.loop(0, CHUNK)
        def _(r):
            @pl.loop(0, C_OUT // 16)
            def _(c):
                rows0[r, pl.ds(c * 16, 16)] = z16

        @pl.loop(0, RPT // CHUNK)
        def _(b):
            pltpu.sync_copy(rows0,
                            acc.at[pl.ds(sid * RPT + b * CHUNK, CHUNK)])
        plsc.subcore_barrier()

        def gather(buf, j):
            pltpu.sync_copy(y_hbm.at[idx_s.at[j]], buf)

        def scatter_start(buf, j, sem):
            pltpu.async_copy(buf, acc.at[idx_d.at[j]], sem, add=True)

        def scatter_wait(buf, j, sem):
            pltpu.make_async_copy(buf, acc.at[idx_d.at[j]], sem).wait()

        for p in range(NJ // HJ):
            base = tid * NJ + p * HJ
            pltpu.sync_copy(src_hbm.at[pl.ds(base, HJ)], idx_s)
            pltpu.sync_copy(dst_hbm.at[pl.ds(base, HJ)], idx_d)

            gather(rows0, 0)
            scatter_start(rows0, 0, ss0)
            gather(rows1, 1)
            scatter_start(rows1, 1, ss1)

            @pl.loop(1, HJ // 2)
            def _(i):
                m = i * 2
                scatter_wait(rows0, m, ss0)
                gather(rows0, m)
                scatter_start(rows0, m, ss0)
                scatter_wait(rows1, m + 1, ss1)
                gather(rows1, m + 1)
                scatter_start(rows1, m + 1, ss1)

            scatter_wait(rows0, 0, ss0)
            scatter_wait(rows1, 1, ss1)

        plsc.subcore_barrier()
        pltpu.sync_copy(
            acc.at[pl.ds(sid * RPT, RPT)],
            out_hbm.at[pl.ds(cid * ACC_ROWS + sid * RPT, RPT)])

    return sc_kernel(y_flat, src2d, dst2d)


def kernel(x, W, edge_index):
    src = edge_index[0].reshape(KV, EPK)
    dst = edge_index[1].reshape(KV, EPK)
    pad = EPK_PAD - EPK
    offs = (jnp.arange(KV, dtype=jnp.int32) * N)[:, None]
    src_pad = jnp.broadcast_to(jnp.arange(pad, dtype=jnp.int32), (KV, pad))
    src_adj = (jnp.concatenate([src, src_pad], axis=1)
               + offs).reshape(NCH, CHUNK)
    dst_p = jnp.pad(dst, ((0, 0), (0, pad)),
                    constant_values=N).reshape(NCH, CHUNK)
    row_pad = NCHP - NCH
    lane = jnp.arange(CHUNK, dtype=jnp.int32)
    src2d = jnp.concatenate(
        [src_adj, jnp.broadcast_to(lane, (row_pad, CHUNK))], axis=0)
    dst2d = jnp.concatenate(
        [dst_p, jnp.broadcast_to(N + lane, (row_pad, CHUNK))], axis=0)
    dst2d = jnp.where(dst2d == N, N + lane[None, :], dst2d)

    y = _all_offset_matmul(x.astype(jnp.bfloat16),
                           W.astype(jnp.bfloat16)).reshape(KV * N, C_OUT)
    part = _sc_gather_scatter(y, src2d, dst2d)
    return _combine(part[:N], part[ACC_ROWS:ACC_ROWS + N])


# speedup vs baseline: 1.2374x; 1.2374x over previous
import functools

import jax
import jax.numpy as jnp
from jax import lax
from jax.experimental import pallas as pl
from jax.experimental.pallas import tpu as pltpu
from jax.experimental.pallas import tpu_sc as plsc

N = 10000
C_IN = 128
C_OUT = 128
KV = 27
E = 324000
EPK = E // KV

CHUNK = 128
CPK = 94
EPK_PAD = CPK * CHUNK
NCH = KV * CPK

NC = 2
NS = 16
NW = NC * NS
NJ = 80
HJ = NJ // 2
NCHP = NW * NJ

ACC_ROWS = 10240
RPT = ACC_ROWS // NS


def _matmul_body(x_ref, w_ref, y_ref):
    y_ref[0] = jnp.dot(x_ref[...], w_ref[0],
                       preferred_element_type=jnp.float32)


def _all_offset_matmul(x, W):
    return pl.pallas_call(
        _matmul_body,
        grid=(KV,),
        in_specs=[
            pl.BlockSpec((N, C_IN), lambda k: (0, 0)),
            pl.BlockSpec((1, C_IN, C_OUT), lambda k: (k, 0, 0)),
        ],
        out_specs=pl.BlockSpec((1, N, C_OUT), lambda k: (k, 0, 0)),
        out_shape=jax.ShapeDtypeStruct((KV, N, C_OUT), jnp.float32),
    )(x, W)


def _combine_body(a_ref, b_ref, o_ref):
    o_ref[...] = a_ref[...] + b_ref[...]


def _combine(p0, p1):
    rb = 2000
    return pl.pallas_call(
        _combine_body,
        grid=(N // rb,),
        in_specs=[
            pl.BlockSpec((rb, C_OUT), lambda r: (r, 0)),
            pl.BlockSpec((rb, C_OUT), lambda r: (r, 0)),
        ],
        out_specs=pl.BlockSpec((rb, C_OUT), lambda r: (r, 0)),
        out_shape=jax.ShapeDtypeStruct((N, C_OUT), jnp.float32),
    )(p0, p1)


def _sc_gather_scatter(y_flat, src2d, dst2d):
    mesh = plsc.VectorSubcoreMesh(core_axis_name="c", subcore_axis_name="s")

    @functools.partial(
        pl.kernel,
        mesh=mesh,
        out_type=jax.ShapeDtypeStruct((NC * ACC_ROWS, C_OUT), jnp.float32),
        scratch_types=[
            pltpu.VMEM((HJ, CHUNK), jnp.int32),
            pltpu.VMEM((HJ, CHUNK), jnp.int32),
            pltpu.VMEM((CHUNK, C_OUT), jnp.float32),
            pltpu.VMEM((CHUNK, C_OUT), jnp.float32),
            pltpu.VMEM_SHARED((ACC_ROWS, C_OUT), jnp.float32),
            pltpu.SemaphoreType.DMA,
            pltpu.SemaphoreType.DMA,
        ],
    )
    def sc_kernel(y_hbm, src_hbm, dst_hbm, out_hbm,
                  idx_s, idx_d, rows0, rows1, acc, ss0, ss1):
        cid = lax.axis_index("c")
        sid = lax.axis_index("s")
        tid = cid * NS + sid

        z16 = jnp.zeros((16,), jnp.float32)

        @pl.loop(0, CHUNK)
        def _(r):
            @pl.loop(0, C_OUT // 16)
            def _(c):
                rows0[r, pl.ds(c * 16, 16)] = z16

        @pl.loop(0, RPT // CHUNK)
        def _(b):
            pltpu.sync_copy(rows0,
                            acc.at[pl.ds(sid * RPT + b * CHUNK, CHUNK)])
        plsc.subcore_barrier()

        def gather(buf, j):
            pltpu.sync_copy(y_hbm.at[idx_s.at[j]], buf)

        def scatter_start(buf, j, sem):
            pltpu.async_copy(buf, acc.at[idx_d.at[j]], sem, add=True)

        def scatter_wait(buf, j, sem):
            pltpu.make_async_copy(buf, acc.at[idx_d.at[j]], sem).wait()

        for p in range(NJ // HJ):
            base = tid * NJ + p * HJ
            pltpu.sync_copy(src_hbm.at[pl.ds(base, HJ)], idx_s)
            pltpu.sync_copy(dst_hbm.at[pl.ds(base, HJ)], idx_d)

            gather(rows0, 0)
            scatter_start(rows0, 0, ss0)
            gather(rows1, 1)
            scatter_start(rows1, 1, ss1)

            @pl.loop(1, HJ // 2)
            def _(i):
                m = i * 2
                scatter_wait(rows0, m, ss0)
                gather(rows0, m)
                scatter_start(rows0, m, ss0)
                scatter_wait(rows1, m + 1, ss1)
                gather(rows1, m + 1)
                scatter_start(rows1, m + 1, ss1)

            scatter_wait(rows0, 0, ss0)
            scatter_wait(rows1, 1, ss1)

        plsc.subcore_barrier()
        pltpu.sync_copy(
            acc.at[pl.ds(sid * RPT, RPT)],
            out_hbm.at[pl.ds(cid * ACC_ROWS + sid * RPT, RPT)])

    return sc_kernel(y_flat, src2d, dst2d)


def kernel(x, W, edge_index):
    src = edge_index[0].reshape(KV, EPK)
    dst = edge_index[1].reshape(KV, EPK)
    pad = EPK_PAD - EPK
    offs = (jnp.arange(KV, dtype=jnp.int32) * N)[:, None]
    src_pad = jnp.broadcast_to(jnp.arange(pad, dtype=jnp.int32), (KV, pad))
    src_adj = (jnp.concatenate([src, src_pad], axis=1)
               + offs).reshape(NCH, CHUNK)
    dst_p = jnp.pad(dst, ((0, 0), (0, pad)),
                    constant_values=N).reshape(NCH, CHUNK)
    row_pad = NCHP - NCH
    lane = jnp.arange(CHUNK, dtype=jnp.int32)
    src2d = jnp.concatenate(
        [src_adj, jnp.broadcast_to(lane, (row_pad, CHUNK))], axis=0)
    dst2d = jnp.concatenate(
        [dst_p, jnp.broadcast_to(N + lane, (row_pad, CHUNK))], axis=0)
    dst2d = jnp.where(dst2d == N, N + lane[None, :], dst2d)

    y = _all_offset_matmul(x.astype(jnp.bfloat16),
                           W.astype(jnp.bfloat16)).reshape(KV * N, C_OUT)
    part = _sc_gather_scatter(y, src2d, dst2d)
    return _combine(part[:N], part[ACC_ROWS:ACC_ROWS + N])
